# baseline (speedup 1.0000x reference)
import functools

import jax
import jax.numpy as jnp
from jax import lax
from jax.experimental import pallas as pl
from jax.experimental.pallas import tpu as pltpu
from jax.experimental.pallas import tpu_sc as plsc

_LANES = 16


def _tec_kernel(n_pts, n_workers, chunk, n_cells,
                x_hbm, cid_hbm, coord_hbm, conn0_hbm, conn1_hbm, out_hbm,
                coord_v, conn0_v, conn1_v, x1t_v, invt_v,
                xb0, xb1, cb0, cb1, ob0, ob1,
                sx0, sx1, sc0, sc1, so0, so1, st0):
    per_worker = n_pts // n_workers
    n_chunks = per_worker // chunk
    wid = lax.axis_index("s") * 2 + lax.axis_index("c")
    base = wid * per_worker

    xb = (xb0, xb1)
    cb = (cb0, cb1)
    ob = (ob0, ob1)
    sx = (sx0, sx1)
    sc = (sc0, sc1)
    so = (so0, so1)

    def start_in(bi, off):
        pltpu.async_copy(x_hbm.at[pl.ds(off, chunk)], xb[bi], sx[bi])
        pltpu.async_copy(cid_hbm.at[pl.ds(off, chunk)], cb[bi], sc[bi])

    def wait_in(bi):
        pltpu.make_async_copy(x_hbm.at[pl.ds(0, chunk)], xb[bi], sx[bi]).wait()
        pltpu.make_async_copy(cid_hbm.at[pl.ds(0, chunk)], cb[bi],
                              sc[bi]).wait()

    def start_out(bi, off):
        pltpu.async_copy(ob[bi], out_hbm.at[pl.ds(2 * off, 2 * chunk)],
                         so[bi])

    def wait_out(bi):
        pltpu.make_async_copy(ob[bi], out_hbm.at[pl.ds(0, 2 * chunk)],
                              so[bi]).wait()

    def compute(bi):
        x_v, cid_v, out_v = xb[bi], cb[bi], ob[bi]

        @plsc.parallel_loop(0, chunk // 128, unroll=4)
        def blk(bk):
            for s in range(128 // _LANES):
                o = bk * 128 + s * _LANES
                cid = cid_v[pl.ds(o, _LANES)]
                x1 = plsc.load_gather(x1t_v, [cid])
                inv = plsc.load_gather(invt_v, [cid])
                xv = x_v[pl.ds(o, _LANES)]
                na = (xv - x1) * inv
                p = bk * 256 + s * _LANES
                out_v[pl.ds(p, _LANES)] = na
                out_v[pl.ds(p + 128, _LANES)] = 1.0 - na

    ht0 = pltpu.async_copy(coord_hbm, coord_v, so0)
    ht1 = pltpu.async_copy(conn0_hbm, conn0_v, so1)
    ht2 = pltpu.async_copy(conn1_hbm, conn1_v, st0)

    start_in(0, base)
    start_in(1, base + chunk)

    ht0.wait()
    ht1.wait()
    ht2.wait()
    for t in range(n_cells // _LANES):
        ds = pl.ds(t * _LANES, _LANES)
        n0 = conn0_v[ds]
        n1 = conn1_v[ds]
        x0 = plsc.load_gather(coord_v, [n0])
        x1 = plsc.load_gather(coord_v, [n1])
        x1t_v[ds] = x1
        invt_v[ds] = 1.0 / (x0 - x1)

    n2 = n_chunks // 2

    def pair_body(g, _):
        for b in range(2):
            off = base + (2 * g + b) * chunk
            wait_in(b)

            @pl.when(g > 0)
            def _drain():
                wait_out(b)

            compute(b)
            start_out(b, off)

            @pl.when(g < n2 - 1)
            def _prefetch():
                start_in(b, off + 2 * chunk)

        return _

    lax.fori_loop(0, n2, pair_body, None)
    wait_out(0)
    wait_out(1)


def _lane_gather(src, idx):
    dn = lax.GatherDimensionNumbers(
        offset_dims=(),
        collapsed_slice_dims=(1,),
        start_index_map=(1,),
        operand_batching_dims=(0,),
        start_indices_batching_dims=(0,),
    )
    return lax.gather(src, idx[..., None], dimension_numbers=dn,
                      slice_sizes=(1, 1),
                      mode=lax.GatherScatterMode.PROMISE_IN_BOUNDS)


def _tc_tail_kernel(_alias_ref, x_ref, cid_ref, coord_ref, c0_ref, c1_ref,
                    out_ref):
    br = x_ref.shape[0]
    coordv = coord_ref[...]
    c0 = c0_ref[...]
    c1 = c1_ref[...]
    x0t = _lane_gather(coordv, c0)
    x1t = _lane_gather(coordv, c1)
    invt = 1.0 / (x0t - x1t)
    cid = cid_ref[...]
    x1e = _lane_gather(jnp.broadcast_to(x1t, (br, 128)), cid)
    inve = _lane_gather(jnp.broadcast_to(invt, (br, 128)), cid)
    na = (x_ref[...] - x1e) * inve
    out_ref[...] = jnp.stack([na, 1.0 - na], axis=1).reshape(2 * br, 128)


def kernel(x, cell_id, coordinates, connectivity):
    n_pts = x.shape[0]
    n_nodes = coordinates.shape[0]
    n_cells = connectivity.shape[0]
    n_workers = 32
    chunk = 8192
    n_sc = n_pts // 2

    coord_flat = coordinates[:, 0]
    conn0 = connectivity[:, 0]
    conn1 = connectivity[:, 1]

    mesh = plsc.VectorSubcoreMesh(core_axis_name="c", subcore_axis_name="s")
    body = functools.partial(_tec_kernel, n_sc, n_workers, chunk, n_cells)
    out_flat = pl.kernel(
        body,
        mesh=mesh,
        out_type=jax.ShapeDtypeStruct((2 * n_pts,), jnp.float32),
        compiler_params=pltpu.CompilerParams(needs_layout_passes=False),
        scratch_types=[
            pltpu.VMEM((n_nodes,), jnp.float32),
            pltpu.VMEM((n_cells,), jnp.int32),
            pltpu.VMEM((n_cells,), jnp.int32),
            pltpu.VMEM((n_cells,), jnp.float32),
            pltpu.VMEM((n_cells,), jnp.float32),
            pltpu.VMEM((chunk,), jnp.float32),
            pltpu.VMEM((chunk,), jnp.float32),
            pltpu.VMEM((chunk,), jnp.int32),
            pltpu.VMEM((chunk,), jnp.int32),
            pltpu.VMEM((2 * chunk,), jnp.float32),
            pltpu.VMEM((2 * chunk,), jnp.float32),
            pltpu.SemaphoreType.DMA,
            pltpu.SemaphoreType.DMA,
            pltpu.SemaphoreType.DMA,
            pltpu.SemaphoreType.DMA,
            pltpu.SemaphoreType.DMA,
            pltpu.SemaphoreType.DMA,
            pltpu.SemaphoreType.DMA,
        ],
    )(x, cell_id, coord_flat, conn0, conn1)

    rows = n_pts // 128
    r0 = n_sc // 128
    br = 512
    grid = (rows - r0) // br
    coordp = jnp.zeros((1, 128), jnp.float32).at[0, :n_nodes].set(coord_flat)
    c0p = jnp.zeros((1, 128), jnp.int32).at[0, :n_cells].set(conn0)
    c1p = jnp.zeros((1, 128), jnp.int32).at[0, :n_cells].set(conn1)
    x2 = x.reshape(rows, 128)
    cid2 = cell_id.reshape(rows, 128)

    out2 = pl.pallas_call(
        _tc_tail_kernel,
        grid=(grid,),
        in_specs=[
            pl.BlockSpec(memory_space=pl.ANY),
            pl.BlockSpec((br, 128), lambda i: (r0 // br + i, 0)),
            pl.BlockSpec((br, 128), lambda i: (r0 // br + i, 0)),
            pl.BlockSpec((1, 128), lambda i: (0, 0)),
            pl.BlockSpec((1, 128), lambda i: (0, 0)),
            pl.BlockSpec((1, 128), lambda i: (0, 0)),
        ],
        out_specs=pl.BlockSpec((2 * br, 128), lambda i: (r0 // br + i, 0)),
        out_shape=jax.ShapeDtypeStruct((2 * rows, 128), jnp.float32),
        input_output_aliases={0: 0},
    )(out_flat.reshape(2 * rows, 128), x2, cid2, coordp, c0p, c1p)

    return out2.reshape(n_pts // 128, 2, 128).transpose(0, 2, 1).reshape(
        n_pts, 2)


# speedup vs baseline: 1.3063x; 1.3063x over previous
import functools

import jax
import jax.numpy as jnp
from jax import lax
from jax.experimental import pallas as pl
from jax.experimental.pallas import tpu as pltpu
from jax.experimental.pallas import tpu_sc as plsc

_LANES = 16


def _tec_kernel(n_pts, n_workers, chunk, n_cells,
                x_hbm, cid_hbm, coord_hbm, conn0_hbm, conn1_hbm, out_hbm,
                coord_v, conn0_v, conn1_v, x1t_v, invt_v,
                xb0, xb1, cb0, cb1, ob0, ob1,
                sx0, sx1, sc0, sc1, so0, so1, st0):
    per_worker = n_pts // n_workers
    n_chunks = per_worker // chunk
    wid = lax.axis_index("s") * 2 + lax.axis_index("c")
    base = wid * per_worker

    xb = (xb0, xb1)
    cb = (cb0, cb1)
    ob = (ob0, ob1)
    sx = (sx0, sx1)
    sc = (sc0, sc1)
    so = (so0, so1)

    def start_in(bi, off):
        pltpu.async_copy(x_hbm.at[pl.ds(off, chunk)], xb[bi], sx[bi])
        pltpu.async_copy(cid_hbm.at[pl.ds(off, chunk)], cb[bi], sc[bi])

    def wait_in(bi):
        pltpu.make_async_copy(x_hbm.at[pl.ds(0, chunk)], xb[bi], sx[bi]).wait()
        pltpu.make_async_copy(cid_hbm.at[pl.ds(0, chunk)], cb[bi],
                              sc[bi]).wait()

    def start_out(bi, off):
        pltpu.async_copy(ob[bi], out_hbm.at[pl.ds(2 * off, 2 * chunk)],
                         so[bi])

    def wait_out(bi):
        pltpu.make_async_copy(ob[bi], out_hbm.at[pl.ds(0, 2 * chunk)],
                              so[bi]).wait()

    def compute(bi):
        x_v, cid_v, out_v = xb[bi], cb[bi], ob[bi]

        @plsc.parallel_loop(0, chunk // 128, unroll=4)
        def blk(bk):
            for s in range(128 // _LANES):
                o = bk * 128 + s * _LANES
                cid = cid_v[pl.ds(o, _LANES)]
                x1 = plsc.load_gather(x1t_v, [cid])
                inv = plsc.load_gather(invt_v, [cid])
                xv = x_v[pl.ds(o, _LANES)]
                na = (xv - x1) * inv
                p = bk * 256 + s * _LANES
                out_v[pl.ds(p, _LANES)] = na
                out_v[pl.ds(p + 128, _LANES)] = 1.0 - na

    ht0 = pltpu.async_copy(coord_hbm, coord_v, so0)
    ht1 = pltpu.async_copy(conn0_hbm, conn0_v, so1)
    ht2 = pltpu.async_copy(conn1_hbm, conn1_v, st0)

    start_in(0, base)
    start_in(1, base + chunk)

    ht0.wait()
    ht1.wait()
    ht2.wait()
    for t in range(n_cells // _LANES):
        ds = pl.ds(t * _LANES, _LANES)
        n0 = conn0_v[ds]
        n1 = conn1_v[ds]
        x0 = plsc.load_gather(coord_v, [n0])
        x1 = plsc.load_gather(coord_v, [n1])
        x1t_v[ds] = x1
        invt_v[ds] = 1.0 / (x0 - x1)

    n2 = n_chunks // 2

    def pair_body(g, _):
        for b in range(2):
            off = base + (2 * g + b) * chunk
            wait_in(b)

            @pl.when(g > 0)
            def _drain():
                wait_out(b)

            compute(b)
            start_out(b, off)

            @pl.when(g < n2 - 1)
            def _prefetch():
                start_in(b, off + 2 * chunk)

        return _

    lax.fori_loop(0, n2, pair_body, None)
    wait_out(0)
    wait_out(1)


def kernel(x, cell_id, coordinates, connectivity):
    n_pts = x.shape[0]
    n_nodes = coordinates.shape[0]
    n_cells = connectivity.shape[0]
    n_workers = 32
    chunk = 8192
    n_sc = n_pts

    coord_flat = coordinates[:, 0]
    conn0 = connectivity[:, 0]
    conn1 = connectivity[:, 1]

    mesh = plsc.VectorSubcoreMesh(core_axis_name="c", subcore_axis_name="s")
    body = functools.partial(_tec_kernel, n_sc, n_workers, chunk, n_cells)
    out_flat = pl.kernel(
        body,
        mesh=mesh,
        out_type=jax.ShapeDtypeStruct((2 * n_pts,), jnp.float32),
        compiler_params=pltpu.CompilerParams(needs_layout_passes=False),
        scratch_types=[
            pltpu.VMEM((n_nodes,), jnp.float32),
            pltpu.VMEM((n_cells,), jnp.int32),
            pltpu.VMEM((n_cells,), jnp.int32),
            pltpu.VMEM((n_cells,), jnp.float32),
            pltpu.VMEM((n_cells,), jnp.float32),
            pltpu.VMEM((chunk,), jnp.float32),
            pltpu.VMEM((chunk,), jnp.float32),
            pltpu.VMEM((chunk,), jnp.int32),
            pltpu.VMEM((chunk,), jnp.int32),
            pltpu.VMEM((2 * chunk,), jnp.float32),
            pltpu.VMEM((2 * chunk,), jnp.float32),
            pltpu.SemaphoreType.DMA,
            pltpu.SemaphoreType.DMA,
            pltpu.SemaphoreType.DMA,
            pltpu.SemaphoreType.DMA,
            pltpu.SemaphoreType.DMA,
            pltpu.SemaphoreType.DMA,
            pltpu.SemaphoreType.DMA,
        ],
    )(x, cell_id, coord_flat, conn0, conn1)

    return out_flat.reshape(n_pts // 128, 2, 128).transpose(0, 2, 1).reshape(
        n_pts, 2)
